# baseline (speedup 1.0000x reference)
import functools

import jax
import jax.numpy as jnp
from jax import lax
from jax.experimental import pallas as pl
from jax.experimental.pallas import tpu as pltpu
from jax.experimental.pallas import tpu_sc as plsc

_A = 16
_TCB = 0
_TC_G = 8
_C = 128
_L = 16
_U = 4


def _tc_body(x0, x1, x2, x3, o):
    g = pl.program_id(0)
    s0 = jnp.sum(x0[...], axis=1)
    s1 = jnp.sum(x1[...], axis=1)
    s2 = jnp.sum(x2[...], axis=1)
    s3 = jnp.sum(x3[...], axis=1)
    acc = jnp.concatenate([s0, s1, s2, s3], axis=-1)

    @pl.when(g == 0)
    def _():
        o[...] = acc

    @pl.when(g > 0)
    def _():
        o[...] += acc


def _tc_call(tensors, start_m, num_m):
    B, D = tensors[0].shape[0], tensors[0].shape[2]
    in_specs = [
        pl.BlockSpec(
            (B, t.shape[1] // 16, D),
            lambda g, start_m=start_m: (0, start_m + g, 0),
        )
        for t in tensors
    ]
    return pl.pallas_call(
        _tc_body,
        grid=(num_m,),
        in_specs=in_specs,
        out_specs=pl.BlockSpec((B, 4 * D), lambda g: (0, 0)),
        out_shape=jax.ShapeDtypeStruct((B, 4 * D), jnp.float32),
    )(*tensors)


@functools.lru_cache(maxsize=None)
def _make_sc_kernel(shapes):
    B, D = shapes[0][0], shapes[0][2]
    NT = len(shapes)
    NV = D // _L
    starts, rows_per_core = [], []
    sched = []
    for t, (_, n, _) in enumerate(shapes):
        k = n * (_A + _TCB) // 16
        r = (n - k) // 2
        starts.append(k)
        rows_per_core.append(r)
        off = 0
        while off < r:
            cr = min(_C, r - off)
            sched.append((t, off, cr))
            off += cr

    mesh = plsc.VectorSubcoreMesh(core_axis_name="c", subcore_axis_name="s")

    W = NT * D // 2

    @functools.partial(
        pl.kernel,
        out_type=jax.ShapeDtypeStruct((2, B, NT * D), jnp.float32),
        mesh=mesh,
        scratch_types=[
            pltpu.VMEM((_C, D), jnp.float32),
            pltpu.VMEM((_C, D), jnp.float32),
            pltpu.VMEM((NT * D,), jnp.float32),
            pltpu.VMEM((W,), jnp.float32),
            pltpu.SemaphoreType.DMA,
            pltpu.SemaphoreType.DMA,
        ],
    )
    def sc_sum(x0, x1, x2, x3, tca, out, buf0, buf1, acc, tbuf, sem0, sem1):
        c = lax.axis_index("c")
        s = lax.axis_index("s")
        xs = (x0, x1, x2, x3)
        bufs = (buf0, buf1)
        sems = (sem0, sem1)

        def start_dma(i):
            t, off, cr = sched[i]
            row0 = starts[t] + c * rows_per_core[t] + off
            return pltpu.async_copy(
                xs[t].at[s, pl.ds(row0, cr), :],
                bufs[i % 2].at[pl.ds(0, cr), :],
                sems[i % 2],
            )

        zero = jnp.zeros((_L,), jnp.float32)

        n = len(sched)
        handles = [None] * n
        if n:
            handles[0] = start_dma(0)
        prev_t = -1
        part = None

        def flush(t, vals):
            for j in range(NV):
                acc[pl.ds(t * D + _L * j, _L)] = vals[j]

        for i in range(n):
            if i + 1 < n:
                handles[i + 1] = start_dma(i + 1)
            handles[i].wait()
            t, _off, cr = sched[i]
            buf = bufs[i % 2]

            if t != prev_t:
                if part is not None:
                    flush(prev_t, part)
                part = tuple(zero for _ in range(NV))
                prev_t = t

            def rows_body(r, carry, buf=buf, m=_U):
                for u in range(m):
                    carry = tuple(
                        carry[j] + buf[m * r + u, pl.ds(_L * j, _L)]
                        for j in range(NV)
                    )
                return carry

            nu, rem = cr // _U, cr % _U
            part = lax.fori_loop(0, nu, rows_body, part)
            for u in range(rem):
                part = tuple(
                    part[j] + buf[nu * _U + u, pl.ds(_L * j, _L)]
                    for j in range(NV)
                )
        if part is not None:
            flush(prev_t, part)
        for t in range(NT):
            if rows_per_core[t] == 0:
                flush(t, tuple(zero for _ in range(NV)))

        pltpu.sync_copy(tca.at[s, pl.ds(c * W, W)], tbuf)
        for j in range(W // _L):
            acc[pl.ds(c * W + _L * j, _L)] += tbuf[pl.ds(_L * j, _L)]

        pltpu.sync_copy(acc, out.at[c, s])

    return sc_sum


def kernel(inputs_0, inputs_1, inputs_2, inputs_3):
    tensors = (inputs_0, inputs_1, inputs_2, inputs_3)
    shapes = tuple(t.shape for t in tensors)
    if _A + _TCB >= 16:
        B, D = inputs_0.shape[0], inputs_0.shape[2]
        flat = [t.reshape(t.shape[0] * t.shape[1], D) for t in tensors]

        def _flat_body(x0, x1, x2, x3, o):
            b = pl.program_id(0)
            g = pl.program_id(1)
            acc = jnp.concatenate(
                [jnp.sum(x[...], axis=0) for x in (x0, x1, x2, x3)]
            )[None, :]

            @pl.when(g == 0)
            def _():
                o[pl.ds(b, 1), :] = acc

            @pl.when(g > 0)
            def _():
                o[pl.ds(b, 1), :] += acc

        in_specs = [
            pl.BlockSpec(
                (t.shape[1] // _TC_G, D),
                lambda b, g, G=_TC_G: (b * G + g, 0),
            )
            for t in tensors
        ]
        return pl.pallas_call(
            _flat_body,
            grid=(B, _TC_G),
            in_specs=in_specs,
            out_specs=pl.BlockSpec((B, 4 * D), lambda b, g: (0, 0)),
            out_shape=jax.ShapeDtypeStruct((B, 4 * D), jnp.float32),
        )(*flat)
    tca = _tc_call(tensors, 0, _A)
    sc_out = _make_sc_kernel(shapes)(*tensors, tca)
    res = sc_out[0] + sc_out[1]
    if _TCB > 0:
        res = res + _tc_call(tensors, _A, _TCB)
    return res


# speedup vs baseline: 2.3853x; 2.3853x over previous
import functools

import jax
import jax.numpy as jnp
from jax import lax
from jax.experimental import pallas as pl
from jax.experimental.pallas import tpu as pltpu
from jax.experimental.pallas import tpu_sc as plsc

_A = 16
_TCB = 0
_TC_G = 8
_C = 128
_L = 16
_U = 4


def _tc_body(x0, x1, x2, x3, o):
    g = pl.program_id(0)
    s0 = jnp.sum(x0[...], axis=1)
    s1 = jnp.sum(x1[...], axis=1)
    s2 = jnp.sum(x2[...], axis=1)
    s3 = jnp.sum(x3[...], axis=1)
    acc = jnp.concatenate([s0, s1, s2, s3], axis=-1)

    @pl.when(g == 0)
    def _():
        o[...] = acc

    @pl.when(g > 0)
    def _():
        o[...] += acc


def _tc_call(tensors, start_m, num_m):
    B, D = tensors[0].shape[0], tensors[0].shape[2]
    in_specs = [
        pl.BlockSpec(
            (B, t.shape[1] // 16, D),
            lambda g, start_m=start_m: (0, start_m + g, 0),
        )
        for t in tensors
    ]
    return pl.pallas_call(
        _tc_body,
        grid=(num_m,),
        in_specs=in_specs,
        out_specs=pl.BlockSpec((B, 4 * D), lambda g: (0, 0)),
        out_shape=jax.ShapeDtypeStruct((B, 4 * D), jnp.float32),
    )(*tensors)


@functools.lru_cache(maxsize=None)
def _make_sc_kernel(shapes):
    B, D = shapes[0][0], shapes[0][2]
    NT = len(shapes)
    NV = D // _L
    starts, rows_per_core = [], []
    sched = []
    for t, (_, n, _) in enumerate(shapes):
        k = n * (_A + _TCB) // 16
        r = (n - k) // 2
        starts.append(k)
        rows_per_core.append(r)
        off = 0
        while off < r:
            cr = min(_C, r - off)
            sched.append((t, off, cr))
            off += cr

    mesh = plsc.VectorSubcoreMesh(core_axis_name="c", subcore_axis_name="s")

    W = NT * D // 2

    @functools.partial(
        pl.kernel,
        out_type=jax.ShapeDtypeStruct((2, B, NT * D), jnp.float32),
        mesh=mesh,
        scratch_types=[
            pltpu.VMEM((_C, D), jnp.float32),
            pltpu.VMEM((_C, D), jnp.float32),
            pltpu.VMEM((NT * D,), jnp.float32),
            pltpu.VMEM((W,), jnp.float32),
            pltpu.SemaphoreType.DMA,
            pltpu.SemaphoreType.DMA,
        ],
    )
    def sc_sum(x0, x1, x2, x3, tca, out, buf0, buf1, acc, tbuf, sem0, sem1):
        c = lax.axis_index("c")
        s = lax.axis_index("s")
        xs = (x0, x1, x2, x3)
        bufs = (buf0, buf1)
        sems = (sem0, sem1)

        def start_dma(i):
            t, off, cr = sched[i]
            row0 = starts[t] + c * rows_per_core[t] + off
            return pltpu.async_copy(
                xs[t].at[s, pl.ds(row0, cr), :],
                bufs[i % 2].at[pl.ds(0, cr), :],
                sems[i % 2],
            )

        zero = jnp.zeros((_L,), jnp.float32)

        n = len(sched)
        handles = [None] * n
        if n:
            handles[0] = start_dma(0)
        prev_t = -1
        part = None

        def flush(t, vals):
            for j in range(NV):
                acc[pl.ds(t * D + _L * j, _L)] = vals[j]

        for i in range(n):
            if i + 1 < n:
                handles[i + 1] = start_dma(i + 1)
            handles[i].wait()
            t, _off, cr = sched[i]
            buf = bufs[i % 2]

            if t != prev_t:
                if part is not None:
                    flush(prev_t, part)
                part = tuple(zero for _ in range(NV))
                prev_t = t

            def rows_body(r, carry, buf=buf, m=_U):
                for u in range(m):
                    carry = tuple(
                        carry[j] + buf[m * r + u, pl.ds(_L * j, _L)]
                        for j in range(NV)
                    )
                return carry

            nu, rem = cr // _U, cr % _U
            part = lax.fori_loop(0, nu, rows_body, part)
            for u in range(rem):
                part = tuple(
                    part[j] + buf[nu * _U + u, pl.ds(_L * j, _L)]
                    for j in range(NV)
                )
        if part is not None:
            flush(prev_t, part)
        for t in range(NT):
            if rows_per_core[t] == 0:
                flush(t, tuple(zero for _ in range(NV)))

        pltpu.sync_copy(tca.at[s, pl.ds(c * W, W)], tbuf)
        for j in range(W // _L):
            acc[pl.ds(c * W + _L * j, _L)] += tbuf[pl.ds(_L * j, _L)]

        pltpu.sync_copy(acc, out.at[c, s])

    return sc_sum


def kernel(inputs_0, inputs_1, inputs_2, inputs_3):
    tensors = (inputs_0, inputs_1, inputs_2, inputs_3)
    shapes = tuple(t.shape for t in tensors)
    if _A + _TCB >= 16:
        return _tc_call(tensors, 0, 16)
    tca = _tc_call(tensors, 0, _A)
    sc_out = _make_sc_kernel(shapes)(*tensors, tca)
    res = sc_out[0] + sc_out[1]
    if _TCB > 0:
        res = res + _tc_call(tensors, _A, _TCB)
    return res
